# baseline (speedup 1.0000x reference)
import functools

import jax
import jax.numpy as jnp
from jax.experimental import pallas as pl
from jax.experimental.pallas import tpu as pltpu
from jax.experimental.pallas import tpu_sc as plsc

N_DIM = 64
N_RESULT = 16
BATCH = 16384

NC = 2
NS = 16
NW = NC * NS
B_PER_W = BATCH // NW
GCHUNK = 128
N_CHUNKS = B_PER_W // GCHUNK


def _sc_gather_body(bidx_hbm, pidx_hbm, btab_hbm, ptab_hbm,
                    bout_hbm, pout_hbm,
                    bidx_v, pidx_v, brows_v, prows_v, sem):
    wid = jax.lax.axis_index("s") * NC + jax.lax.axis_index("c")
    base = wid * B_PER_W
    pltpu.sync_copy(bidx_hbm.at[pl.ds(base, B_PER_W)], bidx_v)
    pltpu.sync_copy(pidx_hbm.at[pl.ds(base, B_PER_W)], pidx_v)
    copies = []
    for j in range(N_CHUNKS):
        sl = pl.ds(j * GCHUNK, GCHUNK)
        copies.append(pltpu.async_copy(
            btab_hbm.at[bidx_v.at[sl]], brows_v.at[sl], sem))
        copies.append(pltpu.async_copy(
            ptab_hbm.at[pidx_v.at[sl]], prows_v.at[sl], sem))
    for c in copies:
        c.wait()
    out_sl = pl.ds(base, B_PER_W)
    pltpu.sync_copy(brows_v, bout_hbm.at[out_sl])
    pltpu.sync_copy(prows_v, pout_hbm.at[out_sl])


@jax.jit
def _sc_gather(bidx, pidx, btab, ptab):
    mesh = plsc.VectorSubcoreMesh(
        core_axis_name="c", subcore_axis_name="s",
        num_cores=NC, num_subcores=NS)
    return pl.kernel(
        _sc_gather_body,
        out_type=(
            jax.ShapeDtypeStruct((BATCH, N_DIM), jnp.float32),
            jax.ShapeDtypeStruct((BATCH, N_DIM), jnp.float32),
        ),
        mesh=mesh,
        scratch_types=[
            pltpu.VMEM((B_PER_W,), jnp.int32),
            pltpu.VMEM((B_PER_W,), jnp.int32),
            pltpu.VMEM((B_PER_W, N_DIM), jnp.float32),
            pltpu.VMEM((B_PER_W, N_DIM), jnp.float32),
            pltpu.SemaphoreType.DMA,
        ],
        compiler_params=pltpu.CompilerParams(use_tc_tiling_on_sc=False),
    )(bidx, pidx, btab, ptab)


def _mm_body(be_ref, pe_ref, w1_ref, w2_ref, b_ref, out_ref):
    acc = jnp.dot(be_ref[...], w1_ref[...], preferred_element_type=jnp.float32)
    acc += jnp.dot(pe_ref[...], w2_ref[...], preferred_element_type=jnp.float32)
    out_ref[...] = acc + b_ref[...]


MM_BLOCK = 2048


@jax.jit
def _tc_linear(be, pe, w1t, w2t, b2d):
    grid = (BATCH // MM_BLOCK,)
    return pl.pallas_call(
        _mm_body,
        grid=grid,
        in_specs=[
            pl.BlockSpec((MM_BLOCK, N_DIM), lambda i: (i, 0)),
            pl.BlockSpec((MM_BLOCK, N_DIM), lambda i: (i, 0)),
            pl.BlockSpec((N_DIM, N_RESULT), lambda i: (0, 0)),
            pl.BlockSpec((N_DIM, N_RESULT), lambda i: (0, 0)),
            pl.BlockSpec((1, N_RESULT), lambda i: (0, 0)),
        ],
        out_specs=pl.BlockSpec((MM_BLOCK, N_RESULT), lambda i: (i, 0)),
        out_shape=jax.ShapeDtypeStruct((BATCH, N_RESULT), jnp.float32),
    )(be, pe, w1t, w2t, b2d)


def kernel(x, batter_table, pitcher_table, W, b):
    bidx = x[:, 0]
    pidx = x[:, 1]
    be, pe = _sc_gather(bidx, pidx, batter_table, pitcher_table)
    w1t = W[:, :N_DIM].T
    w2t = W[:, N_DIM:].T
    y = _tc_linear(be, pe, w1t, w2t, b.reshape(1, N_RESULT))
    return y, be, pe


# speedup vs baseline: 1.9032x; 1.9032x over previous
import functools

import jax
import jax.numpy as jnp
from jax import lax
from jax.experimental import pallas as pl
from jax.experimental.pallas import tpu as pltpu
from jax.experimental.pallas import tpu_sc as plsc

N_ENT = 100000
N_DIM = 64
N_RESULT = 16
BATCH = 16384

NC = 2
NS = 16
NW = NC * NS
C_PER_W = N_DIM // NW
OUT_CHUNK = 8192


def _gather_one_row(idx_v, row_v, out_v):
    def body(i, _):
        idx16 = idx_v[pl.ds(i * 16, 16)]
        out_v[pl.ds(i * 16, 16)] = plsc.load_gather(row_v, [idx16])
        return ()

    lax.fori_loop(0, OUT_CHUNK // 16, body, (), unroll=8)


def _sc_body(btabT_hbm, ptabT_hbm, bidx_hbm, pidx_hbm,
             beT_hbm, peT_hbm,
             idx_v, row_v, out_v):
    wid = lax.axis_index("s") * NC + lax.axis_index("c")
    for tab_hbm, idx_hbm, out_hbm in (
        (btabT_hbm, bidx_hbm, beT_hbm),
        (ptabT_hbm, pidx_hbm, peT_hbm),
    ):
        for k in range(C_PER_W):
            c = wid * C_PER_W + k
            pltpu.sync_copy(tab_hbm.at[c], row_v)
            for j in range(BATCH // OUT_CHUNK):
                pltpu.sync_copy(idx_hbm.at[pl.ds(j * OUT_CHUNK, OUT_CHUNK)],
                                idx_v)
                _gather_one_row(idx_v, row_v, out_v)
                pltpu.sync_copy(out_v,
                                out_hbm.at[c, pl.ds(j * OUT_CHUNK, OUT_CHUNK)])


@jax.jit
def _sc_gather_t(btabT, ptabT, bidx, pidx):
    mesh = plsc.VectorSubcoreMesh(
        core_axis_name="c", subcore_axis_name="s",
        num_cores=NC, num_subcores=NS)
    return pl.kernel(
        _sc_body,
        out_type=(
            jax.ShapeDtypeStruct((N_DIM, BATCH), jnp.float32),
            jax.ShapeDtypeStruct((N_DIM, BATCH), jnp.float32),
        ),
        mesh=mesh,
        scratch_types=[
            pltpu.VMEM((OUT_CHUNK,), jnp.int32),
            pltpu.VMEM((N_ENT,), jnp.float32),
            pltpu.VMEM((OUT_CHUNK,), jnp.float32),
        ],
        compiler_params=pltpu.CompilerParams(
            use_tc_tiling_on_sc=True, needs_layout_passes=False),
    )(btabT, ptabT, bidx, pidx)


def _mm_body(w1_ref, w2_ref, b_ref, beT_ref, peT_ref, out_ref):
    acc = jnp.dot(w1_ref[...], beT_ref[...], preferred_element_type=jnp.float32)
    acc += jnp.dot(w2_ref[...], peT_ref[...], preferred_element_type=jnp.float32)
    out_ref[...] = acc + b_ref[...]


MM_BLOCK = 4096


@jax.jit
def _tc_linear_t(w1, w2, b2d, beT, peT):
    grid = (BATCH // MM_BLOCK,)
    return pl.pallas_call(
        _mm_body,
        grid=grid,
        in_specs=[
            pl.BlockSpec((N_RESULT, N_DIM), lambda i: (0, 0)),
            pl.BlockSpec((N_RESULT, N_DIM), lambda i: (0, 0)),
            pl.BlockSpec((N_RESULT, 1), lambda i: (0, 0)),
            pl.BlockSpec((N_DIM, MM_BLOCK), lambda i: (0, i)),
            pl.BlockSpec((N_DIM, MM_BLOCK), lambda i: (0, i)),
        ],
        out_specs=pl.BlockSpec((N_RESULT, MM_BLOCK), lambda i: (0, i)),
        out_shape=jax.ShapeDtypeStruct((N_RESULT, BATCH), jnp.float32),
    )(w1, w2, b2d, beT, peT)


def kernel(x, batter_table, pitcher_table, W, b):
    bidx = x[:, 0]
    pidx = x[:, 1]
    beT, peT = _sc_gather_t(batter_table.T, pitcher_table.T, bidx, pidx)
    w1 = W[:, :N_DIM]
    w2 = W[:, N_DIM:]
    yT = _tc_linear_t(w1, w2, b.reshape(N_RESULT, 1), beT, peT)
    return yT.T, beT.T, peT.T
